# Initial kernel scaffold; baseline (speedup 1.0000x reference)
import jax
import jax.numpy as jnp
from jax.experimental import pallas as pl


def kernel(nodes, edges, receivers, W_node, b_node, W_edge, b_edge, W_msg, b_msg, W_upd, b_upd, W_att, b_att, W_o1, b_o1, W_o2, b_o2, W_o3, b_o3):
    raise NotImplementedError("write your pallas kernel here")


# speedup vs baseline: 5.2018x; 5.2018x over previous
import functools

import jax
import jax.numpy as jnp
from jax import lax
from jax.experimental import pallas as pl
from jax.experimental.pallas import tpu as pltpu
from jax.experimental.pallas import tpu_sc as plsc

_NC = 2
_NS = 16
_NW = _NC * _NS
_L = 16


def _sc_reduce_body(epw, edges_hbm, rec4_hbm, out_hbm, ev, rv, ov):
    wid = lax.axis_index("s") * _NC + lax.axis_index("c")
    base = wid * epw * 4
    pltpu.sync_copy(edges_hbm.at[pl.ds(base, epw * 4)], ev)
    pltpu.sync_copy(rec4_hbm.at[pl.ds(base, epw * 4)], rv)

    zeros = jnp.zeros((_L,), jnp.float32)
    ones = jnp.full((_L,), 1.0, jnp.float32)

    def body(j, carry):
        acc, accm = carry
        vals = ev[pl.ds(j * _L, _L)]
        rec = rv[pl.ds(j * _L, _L)]
        m = rec == 0
        acc = acc + jnp.where(m, vals, zeros)
        accm = accm + jnp.where(m, ones, zeros)
        return acc, accm

    acc, accm = lax.fori_loop(0, epw // 4, body, (zeros, zeros))
    ov[pl.ds(0, _L)] = acc
    ov[pl.ds(_L, _L)] = accm
    pltpu.sync_copy(ov, out_hbm.at[wid])


def _sc_reduce(edges_flat, receivers4):
    e4 = receivers4.shape[0]
    epw = e4 // (4 * _NW)
    mesh = plsc.VectorSubcoreMesh(core_axis_name="c", subcore_axis_name="s")
    fn = functools.partial(
        pl.kernel,
        mesh=mesh,
        out_type=jax.ShapeDtypeStruct((_NW, 2 * _L), jnp.float32),
        scratch_types=[
            pltpu.VMEM((epw * 4,), jnp.float32),
            pltpu.VMEM((epw * 4,), jnp.int32),
            pltpu.VMEM((2 * _L,), jnp.float32),
        ],
    )(functools.partial(_sc_reduce_body, epw))
    return fn(edges_flat, receivers4)


def _lrelu(x):
    return jnp.where(x >= 0, x, 0.01 * x)


def _dlrelu(x):
    return jnp.where(x >= 0, 1.0, 0.01)


def _dense_body(sc_ref, n0_ref, Wn_ref, bn_ref, Wet_ref, be_ref, Wm_ref,
                bm_ref, Wu_ref, bu_ref, Wo1_ref, bo1_ref, Wo2_ref, bo2_ref,
                Wo3_ref, wo3r_ref, bo3_ref, h_ref, g_ref):
    f32 = jnp.float32
    sc = sc_ref[...]
    colsum = jnp.sum(sc[:, :_L], axis=0, keepdims=True)
    cnt0 = jnp.sum(sc[:, _L:], keepdims=True) * 0.25

    S0 = jnp.dot(colsum, Wet_ref[...], preferred_element_type=f32) \
        + cnt0 * be_ref[...]
    emb0 = jnp.dot(n0_ref[...], Wn_ref[...], preferred_element_type=f32) \
        + bn_ref[...]

    bm = bm_ref[...]
    bu = bu_ref[...]
    e = emb0
    for i in range(Wm_ref.shape[0]):
        agg = jnp.dot(S0, Wm_ref[i], preferred_element_type=f32) \
            + cnt0 * lax.slice(bm, (i, 0), (i + 1, bm.shape[1]))
        x = jnp.concatenate([e, agg], axis=1)
        e = _lrelu(jnp.dot(x, Wu_ref[i], preferred_element_type=f32)
                   + lax.slice(bu, (i, 0), (i + 1, bu.shape[1])))

    Wo1 = Wo1_ref[...]
    bo1 = bo1_ref[...]
    Wo2 = Wo2_ref[...]
    bo2 = bo2_ref[...]
    z1 = jnp.dot(e, Wo1, preferred_element_type=f32) + bo1
    a1 = _lrelu(z1)
    z2 = jnp.dot(a1, Wo2, preferred_element_type=f32) + bo2
    a2 = _lrelu(z2)
    h_ref[...] = jnp.dot(a2, Wo3_ref[...], preferred_element_type=f32) \
        + bo3_ref[...]

    z1g = jnp.dot(emb0, Wo1, preferred_element_type=f32) + bo1
    a1g = _lrelu(z1g)
    z2g = jnp.dot(a1g, Wo2, preferred_element_type=f32) + bo2
    d_z2 = wo3r_ref[...] * _dlrelu(z2g)
    d_a1 = lax.dot_general(d_z2, Wo2, (((1,), (1,)), ((), ())),
                           preferred_element_type=f32)
    d_z1 = d_a1 * _dlrelu(z1g)
    d_emb = lax.dot_general(d_z1, Wo1, (((1,), (1,)), ((), ())),
                            preferred_element_type=f32)
    g_ref[...] = lax.dot_general(d_emb, Wn_ref[...], (((1,), (1,)), ((), ())),
                                 preferred_element_type=f32)


def kernel(nodes, edges, receivers, W_node, b_node, W_edge, b_edge, W_msg,
           b_msg, W_upd, b_upd, W_att, b_att, W_o1, b_o1, W_o2, b_o2, W_o3,
           b_o3):
    f32 = jnp.float32
    sc_out = _sc_reduce(edges.reshape(-1), jnp.repeat(receivers, 4))

    h2, g2 = pl.pallas_call(
        _dense_body,
        out_shape=[
            jax.ShapeDtypeStruct((1, 1), f32),
            jax.ShapeDtypeStruct((1, nodes.shape[1]), f32),
        ],
    )(
        sc_out,
        nodes[0:1, :],
        W_node,
        b_node.reshape(1, -1),
        jnp.tile(W_edge, (4, 1)),
        b_edge.reshape(1, -1),
        W_msg,
        b_msg,
        W_upd,
        b_upd,
        W_o1,
        b_o1.reshape(1, -1),
        W_o2,
        b_o2.reshape(1, -1),
        W_o3,
        W_o3.reshape(1, -1),
        b_o3.reshape(1, -1),
    )
    return (h2[0, 0], g2[0, 3:6])


# speedup vs baseline: 72.1595x; 13.8719x over previous
import functools

import jax
import jax.numpy as jnp
from jax import lax
from jax.experimental import pallas as pl
from jax.experimental.pallas import tpu as pltpu
from jax.experimental.pallas import tpu_sc as plsc

_NC = 2
_NS = 16
_NW = _NC * _NS
_L = 16
_D = 4


def _sc_reduce_body(epw, e_total, ep_hbm, rec_hbm, out_hbm, ev, rv, ov):
    wid = lax.axis_index("s") * _NC + lax.axis_index("c")
    base = wid * epw
    for c in range(_D):
        pltpu.sync_copy(ep_hbm.at[pl.ds(c * e_total + base, epw)],
                        ev.at[pl.ds(c * epw, epw)])
    pltpu.sync_copy(rec_hbm.at[pl.ds(base, epw)], rv)

    zeros = jnp.zeros((_L,), jnp.float32)
    ones = jnp.full((_L,), 1.0, jnp.float32)

    def chunk(j, carry):
        accs = list(carry)
        rec = rv[pl.ds(j * _L, _L)]
        mf = jnp.where(rec == 0, ones, zeros)
        accs[_D] = accs[_D] + mf
        for c in range(_D):
            vals = ev[pl.ds(c * epw + j * _L, _L)]
            accs[c] = accs[c] + vals * mf
        return tuple(accs)

    accs = lax.fori_loop(0, epw // _L, chunk, (zeros,) * (_D + 1))
    for c in range(_D + 1):
        ov[pl.ds(c * _L, _L)] = accs[c]
    pltpu.sync_copy(ov, out_hbm.at[wid])


def _sc_reduce(edges_planar, receivers):
    e = receivers.shape[0]
    epw = e // _NW
    mesh = plsc.VectorSubcoreMesh(core_axis_name="c", subcore_axis_name="s")
    fn = functools.partial(
        pl.kernel,
        mesh=mesh,
        out_type=jax.ShapeDtypeStruct((_NW, (_D + 1) * _L), jnp.float32),
        scratch_types=[
            pltpu.VMEM((_D * epw,), jnp.float32),
            pltpu.VMEM((epw,), jnp.int32),
            pltpu.VMEM(((_D + 1) * _L,), jnp.float32),
        ],
    )(functools.partial(_sc_reduce_body, epw, e))
    return fn(edges_planar, receivers)


def _lrelu(x):
    return jnp.where(x >= 0, x, 0.01 * x)


def _dlrelu(x):
    return jnp.where(x >= 0, 1.0, 0.01)


def _dense_body(sc_ref, n0_ref, Wn_ref, bn_ref, We_ref, be_ref, Wm_ref,
                bm_ref, Wu_ref, bu_ref, Wo1_ref, bo1_ref, Wo2_ref, bo2_ref,
                Wo3_ref, wo3r_ref, bo3_ref, h_ref, g_ref):
    f32 = jnp.float32
    sc = sc_ref[...]
    colsum = jnp.sum(sc, axis=0, keepdims=True)
    parts = [
        jnp.sum(lax.slice(colsum, (0, c * _L), (1, (c + 1) * _L)),
                axis=1, keepdims=True)
        for c in range(_D + 1)
    ]
    T0 = jnp.concatenate(parts[:_D], axis=1)
    cnt0 = parts[_D]

    S0 = jnp.dot(T0, We_ref[...], preferred_element_type=f32) \
        + cnt0 * be_ref[...]
    emb0 = jnp.dot(n0_ref[...], Wn_ref[...], preferred_element_type=f32) \
        + bn_ref[...]

    bm = bm_ref[...]
    bu = bu_ref[...]
    e = emb0
    for i in range(Wm_ref.shape[0]):
        agg = jnp.dot(S0, Wm_ref[i], preferred_element_type=f32) \
            + cnt0 * lax.slice(bm, (i, 0), (i + 1, bm.shape[1]))
        x = jnp.concatenate([e, agg], axis=1)
        e = _lrelu(jnp.dot(x, Wu_ref[i], preferred_element_type=f32)
                   + lax.slice(bu, (i, 0), (i + 1, bu.shape[1])))

    Wo1 = Wo1_ref[...]
    bo1 = bo1_ref[...]
    Wo2 = Wo2_ref[...]
    bo2 = bo2_ref[...]
    z1 = jnp.dot(e, Wo1, preferred_element_type=f32) + bo1
    a1 = _lrelu(z1)
    z2 = jnp.dot(a1, Wo2, preferred_element_type=f32) + bo2
    a2 = _lrelu(z2)
    h_ref[...] = jnp.dot(a2, Wo3_ref[...], preferred_element_type=f32) \
        + bo3_ref[...]

    z1g = jnp.dot(emb0, Wo1, preferred_element_type=f32) + bo1
    a1g = _lrelu(z1g)
    z2g = jnp.dot(a1g, Wo2, preferred_element_type=f32) + bo2
    d_z2 = wo3r_ref[...] * _dlrelu(z2g)
    d_a1 = lax.dot_general(d_z2, Wo2, (((1,), (1,)), ((), ())),
                           preferred_element_type=f32)
    d_z1 = d_a1 * _dlrelu(z1g)
    d_emb = lax.dot_general(d_z1, Wo1, (((1,), (1,)), ((), ())),
                            preferred_element_type=f32)
    g_ref[...] = lax.dot_general(d_emb, Wn_ref[...], (((1,), (1,)), ((), ())),
                                 preferred_element_type=f32)


def kernel(nodes, edges, receivers, W_node, b_node, W_edge, b_edge, W_msg,
           b_msg, W_upd, b_upd, W_att, b_att, W_o1, b_o1, W_o2, b_o2, W_o3,
           b_o3):
    f32 = jnp.float32
    edges_planar = jnp.transpose(edges).reshape(-1)
    sc_out = _sc_reduce(edges_planar, receivers)

    h2, g2 = pl.pallas_call(
        _dense_body,
        out_shape=[
            jax.ShapeDtypeStruct((1, 1), f32),
            jax.ShapeDtypeStruct((1, nodes.shape[1]), f32),
        ],
    )(
        sc_out,
        nodes[0:1, :],
        W_node,
        b_node.reshape(1, -1),
        W_edge,
        b_edge.reshape(1, -1),
        W_msg,
        b_msg,
        W_upd,
        b_upd,
        W_o1,
        b_o1.reshape(1, -1),
        W_o2,
        b_o2.reshape(1, -1),
        W_o3,
        W_o3.reshape(1, -1),
        b_o3.reshape(1, -1),
    )
    return (h2[0, 0], g2[0, 3:6])


# speedup vs baseline: 72.8680x; 1.0098x over previous
import functools

import jax
import jax.numpy as jnp
from jax import lax
from jax.experimental import pallas as pl
from jax.experimental.pallas import tpu as pltpu
from jax.experimental.pallas import tpu_sc as plsc

_NC = 2
_NS = 16
_NW = _NC * _NS
_L = 16
_D = 4


def _sc_reduce_body(epw, e_total, ep_hbm, rec_hbm, out_hbm, ev, rv, ov):
    wid = lax.axis_index("s") * _NC + lax.axis_index("c")
    base = wid * epw
    for c in range(_D):
        pltpu.sync_copy(ep_hbm.at[pl.ds(c * e_total + base, epw)],
                        ev.at[pl.ds(c * epw, epw)])
    pltpu.sync_copy(rec_hbm.at[pl.ds(base, epw)], rv)

    zeros = jnp.zeros((_L,), jnp.float32)
    ones = jnp.full((_L,), 1.0, jnp.float32)

    unroll = 5
    assert (epw // _L) % unroll == 0

    def chunk(j, carry):
        accs = list(carry)
        for u in range(unroll):
            jj = j * unroll + u
            rec = rv[pl.ds(jj * _L, _L)]
            mf = jnp.where(rec == 0, ones, zeros)
            accs[_D] = accs[_D] + mf
            for c in range(_D):
                vals = ev[pl.ds(c * epw + jj * _L, _L)]
                accs[c] = accs[c] + vals * mf
        return tuple(accs)

    accs = lax.fori_loop(0, epw // _L // unroll, chunk, (zeros,) * (_D + 1))
    for c in range(_D + 1):
        ov[pl.ds(c * _L, _L)] = accs[c]
    pltpu.sync_copy(ov, out_hbm.at[wid])


def _sc_reduce(edges_planar, receivers):
    e = receivers.shape[0]
    epw = e // _NW
    mesh = plsc.VectorSubcoreMesh(core_axis_name="c", subcore_axis_name="s")
    fn = functools.partial(
        pl.kernel,
        mesh=mesh,
        out_type=jax.ShapeDtypeStruct((_NW, (_D + 1) * _L), jnp.float32),
        scratch_types=[
            pltpu.VMEM((_D * epw,), jnp.float32),
            pltpu.VMEM((epw,), jnp.int32),
            pltpu.VMEM(((_D + 1) * _L,), jnp.float32),
        ],
    )(functools.partial(_sc_reduce_body, epw, e))
    return fn(edges_planar, receivers)


def _lrelu(x):
    return jnp.where(x >= 0, x, 0.01 * x)


def _dlrelu(x):
    return jnp.where(x >= 0, 1.0, 0.01)


def _dense_body(sc_ref, n0_ref, Wn_ref, bn_ref, We_ref, be_ref, Wm_ref,
                bm_ref, Wu_ref, bu_ref, Wo1_ref, bo1_ref, Wo2_ref, bo2_ref,
                Wo3_ref, wo3r_ref, bo3_ref, h_ref, g_ref):
    f32 = jnp.float32
    sc = sc_ref[...]
    colsum = jnp.sum(sc, axis=0, keepdims=True)
    parts = [
        jnp.sum(lax.slice(colsum, (0, c * _L), (1, (c + 1) * _L)),
                axis=1, keepdims=True)
        for c in range(_D + 1)
    ]
    T0 = jnp.concatenate(parts[:_D], axis=1)
    cnt0 = parts[_D]

    S0 = jnp.dot(T0, We_ref[...], preferred_element_type=f32) \
        + cnt0 * be_ref[...]
    emb0 = jnp.dot(n0_ref[...], Wn_ref[...], preferred_element_type=f32) \
        + bn_ref[...]

    bm = bm_ref[...]
    bu = bu_ref[...]
    e = emb0
    for i in range(Wm_ref.shape[0]):
        agg = jnp.dot(S0, Wm_ref[i], preferred_element_type=f32) \
            + cnt0 * lax.slice(bm, (i, 0), (i + 1, bm.shape[1]))
        x = jnp.concatenate([e, agg], axis=1)
        e = _lrelu(jnp.dot(x, Wu_ref[i], preferred_element_type=f32)
                   + lax.slice(bu, (i, 0), (i + 1, bu.shape[1])))

    Wo1 = Wo1_ref[...]
    bo1 = bo1_ref[...]
    Wo2 = Wo2_ref[...]
    bo2 = bo2_ref[...]
    z1 = jnp.dot(e, Wo1, preferred_element_type=f32) + bo1
    a1 = _lrelu(z1)
    z2 = jnp.dot(a1, Wo2, preferred_element_type=f32) + bo2
    a2 = _lrelu(z2)
    h_ref[...] = jnp.dot(a2, Wo3_ref[...], preferred_element_type=f32) \
        + bo3_ref[...]

    z1g = jnp.dot(emb0, Wo1, preferred_element_type=f32) + bo1
    a1g = _lrelu(z1g)
    z2g = jnp.dot(a1g, Wo2, preferred_element_type=f32) + bo2
    d_z2 = wo3r_ref[...] * _dlrelu(z2g)
    d_a1 = lax.dot_general(d_z2, Wo2, (((1,), (1,)), ((), ())),
                           preferred_element_type=f32)
    d_z1 = d_a1 * _dlrelu(z1g)
    d_emb = lax.dot_general(d_z1, Wo1, (((1,), (1,)), ((), ())),
                            preferred_element_type=f32)
    g_ref[...] = lax.dot_general(d_emb, Wn_ref[...], (((1,), (1,)), ((), ())),
                                 preferred_element_type=f32)


def kernel(nodes, edges, receivers, W_node, b_node, W_edge, b_edge, W_msg,
           b_msg, W_upd, b_upd, W_att, b_att, W_o1, b_o1, W_o2, b_o2, W_o3,
           b_o3):
    f32 = jnp.float32
    edges_planar = jnp.transpose(edges).reshape(-1)
    sc_out = _sc_reduce(edges_planar, receivers)

    h2, g2 = pl.pallas_call(
        _dense_body,
        out_shape=[
            jax.ShapeDtypeStruct((1, 1), f32),
            jax.ShapeDtypeStruct((1, nodes.shape[1]), f32),
        ],
    )(
        sc_out,
        nodes[0:1, :],
        W_node,
        b_node.reshape(1, -1),
        W_edge,
        b_edge.reshape(1, -1),
        W_msg,
        b_msg,
        W_upd,
        b_upd,
        W_o1,
        b_o1.reshape(1, -1),
        W_o2,
        b_o2.reshape(1, -1),
        W_o3,
        W_o3.reshape(1, -1),
        b_o3.reshape(1, -1),
    )
    return (h2[0, 0], g2[0, 3:6])


# speedup vs baseline: 78.8407x; 1.0820x over previous
import functools

import jax
import jax.numpy as jnp
from jax import lax
from jax.experimental import pallas as pl
from jax.experimental.pallas import tpu as pltpu
from jax.experimental.pallas import tpu_sc as plsc

_NC = 2
_NS = 16
_NW = _NC * _NS
_L = 16
_D = 4


def _sc_reduce_body(epw, e_total, ep_hbm, rec_hbm, out_hbm, ev, rv, ov, sem):
    wid = lax.axis_index("s") * _NC + lax.axis_index("c")
    base = wid * epw
    cps = [
        pltpu.async_copy(ep_hbm.at[pl.ds(c * e_total + base, epw)],
                         ev.at[pl.ds(c * epw, epw)], sem)
        for c in range(_D)
    ]
    cps.append(pltpu.async_copy(rec_hbm.at[pl.ds(base, epw)], rv, sem))
    for cp in cps:
        cp.wait()

    zeros = jnp.zeros((_L,), jnp.float32)
    ones = jnp.full((_L,), 1.0, jnp.float32)

    unroll = 5
    assert (epw // _L) % unroll == 0

    def chunk(j, carry):
        accs = list(carry)
        for u in range(unroll):
            jj = j * unroll + u
            rec = rv[pl.ds(jj * _L, _L)]
            mf = jnp.where(rec == 0, ones, zeros)
            accs[_D] = accs[_D] + mf
            for c in range(_D):
                vals = ev[pl.ds(c * epw + jj * _L, _L)]
                accs[c] = accs[c] + vals * mf
        return tuple(accs)

    accs = lax.fori_loop(0, epw // _L // unroll, chunk, (zeros,) * (_D + 1))
    for c in range(_D + 1):
        ov[pl.ds(c * _L, _L)] = accs[c]
    pltpu.sync_copy(ov, out_hbm.at[wid])


def _sc_reduce(edges_planar, receivers):
    e = receivers.shape[0]
    epw = e // _NW
    mesh = plsc.VectorSubcoreMesh(core_axis_name="c", subcore_axis_name="s")
    fn = functools.partial(
        pl.kernel,
        mesh=mesh,
        out_type=jax.ShapeDtypeStruct((_NW, (_D + 1) * _L), jnp.float32),
        scratch_types=[
            pltpu.VMEM((_D * epw,), jnp.float32),
            pltpu.VMEM((epw,), jnp.int32),
            pltpu.VMEM(((_D + 1) * _L,), jnp.float32),
            pltpu.SemaphoreType.DMA,
        ],
    )(functools.partial(_sc_reduce_body, epw, e))
    return fn(edges_planar, receivers)


def _lrelu(x):
    return jnp.where(x >= 0, x, 0.01 * x)


def _dlrelu(x):
    return jnp.where(x >= 0, 1.0, 0.01)


def _dense_body(sc_ref, n0_ref, Wn_ref, bn_ref, We_ref, be_ref, Wm_ref,
                bm_ref, Wu_ref, bu_ref, Wo1_ref, bo1_ref, Wo2_ref, bo2_ref,
                Wo3_ref, wo3r_ref, bo3_ref, h_ref, g_ref):
    f32 = jnp.float32
    sc = sc_ref[...]
    colsum = jnp.sum(sc, axis=0, keepdims=True)
    parts = [
        jnp.sum(lax.slice(colsum, (0, c * _L), (1, (c + 1) * _L)),
                axis=1, keepdims=True)
        for c in range(_D + 1)
    ]
    T0 = jnp.concatenate(parts[:_D], axis=1)
    cnt0 = parts[_D]

    S0 = jnp.dot(T0, We_ref[...], preferred_element_type=f32) \
        + cnt0 * be_ref[...]
    emb0 = jnp.dot(n0_ref[...], Wn_ref[...], preferred_element_type=f32) \
        + bn_ref[...]

    bm = bm_ref[...]
    bu = bu_ref[...]
    e = emb0
    for i in range(Wm_ref.shape[0]):
        agg = jnp.dot(S0, Wm_ref[i], preferred_element_type=f32) \
            + cnt0 * lax.slice(bm, (i, 0), (i + 1, bm.shape[1]))
        x = jnp.concatenate([e, agg], axis=1)
        e = _lrelu(jnp.dot(x, Wu_ref[i], preferred_element_type=f32)
                   + lax.slice(bu, (i, 0), (i + 1, bu.shape[1])))

    Wo1 = Wo1_ref[...]
    bo1 = bo1_ref[...]
    Wo2 = Wo2_ref[...]
    bo2 = bo2_ref[...]
    z1 = jnp.dot(e, Wo1, preferred_element_type=f32) + bo1
    a1 = _lrelu(z1)
    z2 = jnp.dot(a1, Wo2, preferred_element_type=f32) + bo2
    a2 = _lrelu(z2)
    h_ref[...] = jnp.dot(a2, Wo3_ref[...], preferred_element_type=f32) \
        + bo3_ref[...]

    z1g = jnp.dot(emb0, Wo1, preferred_element_type=f32) + bo1
    a1g = _lrelu(z1g)
    z2g = jnp.dot(a1g, Wo2, preferred_element_type=f32) + bo2
    d_z2 = wo3r_ref[...] * _dlrelu(z2g)
    d_a1 = lax.dot_general(d_z2, Wo2, (((1,), (1,)), ((), ())),
                           preferred_element_type=f32)
    d_z1 = d_a1 * _dlrelu(z1g)
    d_emb = lax.dot_general(d_z1, Wo1, (((1,), (1,)), ((), ())),
                            preferred_element_type=f32)
    g_ref[...] = lax.dot_general(d_emb, Wn_ref[...], (((1,), (1,)), ((), ())),
                                 preferred_element_type=f32)


def kernel(nodes, edges, receivers, W_node, b_node, W_edge, b_edge, W_msg,
           b_msg, W_upd, b_upd, W_att, b_att, W_o1, b_o1, W_o2, b_o2, W_o3,
           b_o3):
    f32 = jnp.float32
    edges_planar = jnp.transpose(edges).reshape(-1)
    sc_out = _sc_reduce(edges_planar, receivers)

    h2, g2 = pl.pallas_call(
        _dense_body,
        out_shape=[
            jax.ShapeDtypeStruct((1, 1), f32),
            jax.ShapeDtypeStruct((1, nodes.shape[1]), f32),
        ],
    )(
        sc_out,
        nodes[0:1, :],
        W_node,
        b_node.reshape(1, -1),
        W_edge,
        b_edge.reshape(1, -1),
        W_msg,
        b_msg,
        W_upd,
        b_upd,
        W_o1,
        b_o1.reshape(1, -1),
        W_o2,
        b_o2.reshape(1, -1),
        W_o3,
        W_o3.reshape(1, -1),
        b_o3.reshape(1, -1),
    )
    return (h2[0, 0], g2[0, 3:6])
